# baseline (speedup 1.0000x reference)
import functools

import jax
import jax.numpy as jnp
from jax import lax
from jax.experimental import pallas as pl
from jax.experimental.pallas import tpu as pltpu
from jax.experimental.pallas import tpu_sc as plsc


@functools.lru_cache(maxsize=None)
def _build(V, D, R, T):
    B = R * T
    info = plsc.get_sparse_core_info()
    NC, NS = info.num_cores, info.num_subcores
    NW = NC * NS
    assert B % NW == 0
    b_per_w = B // NW
    assert T % b_per_w == 0
    w_per_row = T // b_per_w
    C = 8
    NBUF = 2
    K = 1
    assert b_per_w % C == 0
    n_chunks = b_per_w // C
    assert n_chunks % NBUF == 0 and n_chunks >= NBUF

    mesh = plsc.VectorSubcoreMesh(core_axis_name="c", subcore_axis_name="s")

    @functools.partial(
        pl.kernel,
        mesh=mesh,
        out_type=jax.ShapeDtypeStruct((R, T, D), jnp.float32),
        scratch_types=[
            pltpu.VMEM((b_per_w,), jnp.int32),
            pltpu.VMEM((NBUF, C, D), jnp.float32),
            pltpu.SemaphoreType.DMA((NBUF,)),
            pltpu.SemaphoreType.DMA((NBUF,)),
        ],
    )
    def gather_kernel(idx_hbm, table_hbm, out_hbm, idx_v, rows_v, gsem, wsem):
        wid = lax.axis_index("s") * NC + lax.axis_index("c")
        r = wid // w_per_row
        c = (wid % w_per_row) * b_per_w
        pltpu.sync_copy(idx_hbm.at[r, pl.ds(c, b_per_w)], idx_v)

        def fire_gather(chunk, slot):
            pltpu.async_copy(
                table_hbm.at[idx_v.at[pl.ds(chunk * C, C)]],
                rows_v.at[slot],
                gsem.at[slot],
            )

        def wait_gather(slot):
            pltpu.make_async_copy(
                table_hbm.at[idx_v.at[pl.ds(0, C)]], rows_v.at[slot], gsem.at[slot]
            ).wait()

        def fire_write(chunk, slot):
            pltpu.async_copy(
                rows_v.at[slot],
                out_hbm.at[r, pl.ds(c + chunk * C, C)],
                wsem.at[slot],
            )

        def wait_write(slot):
            pltpu.make_async_copy(
                rows_v.at[slot], out_hbm.at[r, pl.ds(c, C)], wsem.at[slot]
            ).wait()

        for b in range(K):
            fire_gather(b, b)

        n_groups = n_chunks // NBUF

        def group(g, carry):
            for b in range(NBUF):
                i = g * NBUF + b
                f_slot = (b + K) % NBUF
                j2 = i + K
                w = j2 - NBUF

                @pl.when((w >= 0) & (j2 < n_chunks))
                def _():
                    wait_write(f_slot)

                @pl.when(j2 < n_chunks)
                def _():
                    fire_gather(j2, f_slot)

                wait_gather(b)
                fire_write(i, b)
            return carry

        lax.fori_loop(0, n_groups, group, 0)

        for t in range(NBUF):
            wait_write((n_chunks - NBUF + t) % NBUF)

    return gather_kernel


def kernel(x, embedding):
    V, D = embedding.shape
    R, T = x.shape
    return _build(V, D, R, T)(x, embedding)


# speedup vs baseline: 1.0121x; 1.0121x over previous
import functools

import jax
import jax.numpy as jnp
from jax import lax
from jax.experimental import pallas as pl
from jax.experimental.pallas import tpu as pltpu
from jax.experimental.pallas import tpu_sc as plsc


@functools.lru_cache(maxsize=None)
def _build(V, D, R, T):
    B = R * T
    info = plsc.get_sparse_core_info()
    NC, NS = info.num_cores, info.num_subcores
    NW = NC * NS
    assert B % NW == 0
    b_per_w = B // NW
    assert T % b_per_w == 0
    w_per_row = T // b_per_w
    C = 8
    NBUF = 3
    K = 2
    assert b_per_w % C == 0
    n_chunks = b_per_w // C
    assert n_chunks >= NBUF

    mesh = plsc.VectorSubcoreMesh(core_axis_name="c", subcore_axis_name="s")

    @functools.partial(
        pl.kernel,
        mesh=mesh,
        out_type=jax.ShapeDtypeStruct((R, T, D), jnp.float32),
        scratch_types=[
            pltpu.VMEM((b_per_w,), jnp.int32),
            pltpu.VMEM((NBUF, C, D), jnp.float32),
            pltpu.SemaphoreType.DMA((NBUF,)),
            pltpu.SemaphoreType.DMA((NBUF,)),
        ],
    )
    def gather_kernel(idx_hbm, table_hbm, out_hbm, idx_v, rows_v, gsem, wsem):
        wid = lax.axis_index("s") * NC + lax.axis_index("c")
        r = wid // w_per_row
        c = (wid % w_per_row) * b_per_w
        pltpu.sync_copy(idx_hbm.at[r, pl.ds(c, b_per_w)], idx_v)

        def fire_gather(chunk, slot):
            pltpu.async_copy(
                table_hbm.at[idx_v.at[pl.ds(chunk * C, C)]],
                rows_v.at[slot],
                gsem.at[slot],
            )

        def wait_gather(slot):
            pltpu.make_async_copy(
                table_hbm.at[idx_v.at[pl.ds(0, C)]], rows_v.at[slot], gsem.at[slot]
            ).wait()

        def fire_write(chunk, slot):
            pltpu.async_copy(
                rows_v.at[slot],
                out_hbm.at[r, pl.ds(c + chunk * C, C)],
                wsem.at[slot],
            )

        def wait_write(slot):
            pltpu.make_async_copy(
                rows_v.at[slot], out_hbm.at[r, pl.ds(c, C)], wsem.at[slot]
            ).wait()

        for b in range(K):
            fire_gather(b, b)

        n_groups = n_chunks // NBUF

        def group(g, carry):
            for b in range(NBUF):
                i = g * NBUF + b
                f_slot = (b + K) % NBUF
                j2 = i + K
                w = j2 - NBUF

                @pl.when((w >= 0) & (j2 < n_chunks))
                def _():
                    wait_write(f_slot)

                @pl.when(j2 < n_chunks)
                def _():
                    fire_gather(j2, f_slot)

                wait_gather(b)
                fire_write(i, b)
            return carry

        lax.fori_loop(0, n_groups, group, 0)

        for i in range(n_groups * NBUF, n_chunks):
            wait_gather(i % NBUF)
            fire_write(i, i % NBUF)

        for t in range(NBUF):
            wait_write((n_chunks - NBUF + t) % NBUF)

    return gather_kernel


def kernel(x, embedding):
    V, D = embedding.shape
    R, T = x.shape
    return _build(V, D, R, T)(x, embedding)
